# baseline (speedup 1.0000x reference)
import functools

import jax
import jax.numpy as jnp
from jax import lax
from jax.experimental import pallas as pl
from jax.experimental.pallas import tpu as pltpu
from jax.experimental.pallas import tpu_sc as plsc

B = 1024
A = 32
F = 41024
H = 512
HL = 128
Q = H // HL

NC = 2
NS = 16
L = 16
NW = NC * NS
RPW = B // NW
NCHUNK = 2 * RPW
GPC = A * Q


def _sc_embed_body(widx, bidx, tab4, out, allidx, gat, dupf, rows0, rows1,
                   stage, sem0, sem1):
    cid = lax.axis_index("c")
    sid = lax.axis_index("s")
    w = cid * NS + sid
    b0 = w * RPW
    lane = lax.iota(jnp.int32, L)

    pltpu.sync_copy(widx.at[pl.ds(b0 * A, RPW * A)], allidx.at[pl.ds(0, RPW * A)])
    pltpu.sync_copy(bidx.at[pl.ds(b0 * A, RPW * A)], allidx.at[pl.ds(RPW * A, RPW * A)])

    def build(it, _):
        g = it >> 1
        s = it & 1
        local_b = g * L + lane
        base = (s * RPW + local_b) * A
        ts = [plsc.load_gather(allidx, [base + p]) for p in range(A)]
        ci = local_b * 2 + s
        one = jnp.full((L,), 1.0, jnp.float32)
        plsc.store_scatter(dupf, [ci * A], one)
        for i in range(1, A):
            m = ts[i] == ts[0]
            for j in range(1, i):
                m = m | (ts[i] == ts[j])
            plsc.store_scatter(dupf, [ci * A + i], jnp.where(m, 0.0, one))
        for a in range(A):
            t4 = ts[a] * Q
            for q in range(Q):
                col = jnp.full((L,), a * Q + q, jnp.int32)
                plsc.store_scatter(gat, [ci, col], t4 + q)
        return 0

    lax.fori_loop(0, 4, build, 0, unroll=False)

    pltpu.async_copy(tab4.at[gat.at[0]], rows0, sem0)

    def reduce_chunk(ci, rows):
        acc = [jnp.zeros((L,), jnp.float32) for _ in range(H // L)]
        for a in range(A):
            f = plsc.load_gather(dupf, [jnp.broadcast_to(ci * A + a, (L,))])
            for q in range(Q):
                r = a * Q + q
                for c in range(HL // L):
                    acc[q * (HL // L) + c] = acc[q * (HL // L) + c] + \
                        rows[r, pl.ds(c * L, L)] * f
        srow = ci * Q
        for q in range(Q):
            for c in range(HL // L):
                v = acc[q * (HL // L) + c]
                v = jnp.minimum(jnp.maximum(v, -1.0), 1.0)
                stage[srow + q, pl.ds(c * L, L)] = v

    def chunk(i, _):
        ci = i * 2
        pltpu.async_copy(tab4.at[gat.at[ci + 1]], rows1, sem1)
        pltpu.make_async_copy(tab4.at[gat.at[ci]], rows0, sem0).wait()
        reduce_chunk(ci, rows0)

        @pl.when(i < NCHUNK // 2 - 1)
        def _():
            pltpu.async_copy(tab4.at[gat.at[ci + 2]], rows0, sem0)

        pltpu.make_async_copy(tab4.at[gat.at[ci + 1]], rows1, sem1).wait()
        reduce_chunk(ci + 1, rows1)
        return 0

    lax.fori_loop(0, NCHUNK // 2, chunk, 0, unroll=False)

    pltpu.sync_copy(stage, out.at[pl.ds(w * NCHUNK * Q, NCHUNK * Q)])


_sc_embed = functools.partial(
    pl.kernel,
    out_type=jax.ShapeDtypeStruct((2 * B * Q, HL), jnp.float32),
    mesh=plsc.VectorSubcoreMesh(
        core_axis_name="c", subcore_axis_name="s", num_cores=NC, num_subcores=NS
    ),
    compiler_params=pltpu.CompilerParams(needs_layout_passes=False),
    scratch_types=[
        pltpu.VMEM((2 * RPW * A,), jnp.int32),
        pltpu.VMEM((NCHUNK, GPC), jnp.int32),
        pltpu.VMEM((NCHUNK * A,), jnp.float32),
        pltpu.VMEM((GPC, HL), jnp.float32),
        pltpu.VMEM((GPC, HL), jnp.float32),
        pltpu.VMEM((NCHUNK * Q, HL), jnp.float32),
        pltpu.SemaphoreType.DMA,
        pltpu.SemaphoreType.DMA,
    ],
)(_sc_embed_body)


def _mlp_body(x_ref, w1, b1, w2, b2, w3, b3, wo, o_ref):
    cdims = (((1,), (1,)), ((), ()))
    h = lax.dot_general(x_ref[...], w1[...], cdims,
                        preferred_element_type=jnp.float32)
    h = jnp.maximum(h + b1[...], 0.0)
    h = lax.dot_general(h, w2[...], cdims, preferred_element_type=jnp.float32)
    h = jnp.maximum(h + b2[...], 0.0)
    h = lax.dot_general(h, w3[...], cdims, preferred_element_type=jnp.float32)
    h = jnp.maximum(h + b3[...], 0.0)
    o_ref[...] = lax.dot_general(h, wo[...], cdims,
                                 preferred_element_type=jnp.float32)


def _mlp(x, W1, b1, W2, b2, W3, b3, W_out):
    BM = 512
    full = lambda i: (0, 0)
    return pl.pallas_call(
        _mlp_body,
        grid=(B // BM,),
        in_specs=[
            pl.BlockSpec((BM, 2 * H), lambda i: (i, 0)),
            pl.BlockSpec((H, 2 * H), full),
            pl.BlockSpec((1, H), full),
            pl.BlockSpec((H // 2, H), full),
            pl.BlockSpec((1, H // 2), full),
            pl.BlockSpec((H // 4, H // 2), full),
            pl.BlockSpec((1, H // 4), full),
            pl.BlockSpec((1, H // 4), full),
        ],
        out_specs=pl.BlockSpec((BM, 1), lambda i: (i, 0)),
        out_shape=jax.ShapeDtypeStruct((B, 1), jnp.float32),
    )(x, W1, b1.reshape(1, H), W2, b2.reshape(1, H // 2),
      W3, b3.reshape(1, H // 4), W_out)


def kernel(white_indices, black_indices, W_ft, W1, b1, W2, b2, W3, b3, W_out, b_out):
    tab4 = W_ft.T.reshape(-1, HL)
    ft = _sc_embed(white_indices.astype(jnp.int32).reshape(-1),
                   black_indices.astype(jnp.int32).reshape(-1), tab4)
    x = ft.reshape(B, 2 * H)
    out = _mlp(x, W1, b1, W2, b2, W3, b3, W_out)
    return out[:, 0] + b_out
